# Initial kernel scaffold; baseline (speedup 1.0000x reference)
import jax
import jax.numpy as jnp
from jax.experimental import pallas as pl


def kernel(xyzs, features, W1, b1, W2, b2):
    raise NotImplementedError("write your pallas kernel here")


# speedup vs baseline: 36.2823x; 36.2823x over previous
import functools

import jax
import jax.numpy as jnp
from jax import lax
from jax.experimental import pallas as pl
from jax.experimental.pallas import tpu as pltpu

B, T, N, C = 4, 4, 1024, 64
K = 16
NP = 512
F32 = jnp.float32


def _fps_body(xt_ref, aidx_ref, dists_ref):
    x = xt_ref[0]
    y = xt_ref[1]
    z = xt_ref[2]
    iota = lax.broadcasted_iota(jnp.int32, (B * T, N), 1).astype(F32)
    step_iota = lax.broadcasted_iota(jnp.int32, (B * T, NP), 1).astype(F32)
    dists_ref[...] = jnp.full((B * T, N), 1e10, F32)
    aidx_ref[...] = jnp.zeros((B * T, NP), F32)

    def step(i, far):
        aidx_ref[...] += jnp.where(step_iota == i.astype(F32), far, 0.0)
        sel = iota == far
        cx = jnp.sum(jnp.where(sel, x, 0.0), axis=1, keepdims=True)
        cy = jnp.sum(jnp.where(sel, y, 0.0), axis=1, keepdims=True)
        cz = jnp.sum(jnp.where(sel, z, 0.0), axis=1, keepdims=True)
        dx = x - cx
        dy = y - cy
        dz = z - cz
        d = (dx * dx + dy * dy) + dz * dz
        dmin = jnp.minimum(dists_ref[...], d)
        dists_ref[...] = dmin
        m = jnp.max(dmin, axis=1, keepdims=True)
        far2 = jnp.min(jnp.where(dmin == m, iota, F32(1e9)), axis=1, keepdims=True)
        return far2

    lax.fori_loop(0, NP, step, jnp.zeros((B * T, 1), F32))


def _run_fps(xyzs):
    xt = jnp.transpose(xyzs.reshape(B * T, N, 3), (2, 0, 1))
    return pl.pallas_call(
        _fps_body,
        out_shape=jax.ShapeDtypeStruct((B * T, NP), F32),
        in_specs=[pl.BlockSpec((3, B * T, N), lambda: (0, 0, 0))],
        out_specs=pl.BlockSpec((B * T, NP), lambda: (0, 0)),
        scratch_shapes=[pltpu.VMEM((B * T, N), F32)],
    )(xt)


def _main_body(xta_ref, xtn_ref, fa_ref, fn_ref, aidx_ref,
               w1x_ref, w1f_ref, w1a_ref, b1_ref, w2_ref, b2_ref,
               nxyz_ref, nfeat_ref):
    xa = xta_ref[0, 0]
    xn = xtn_ref[0, 0]
    f1 = fa_ref[0, 0]
    f2 = fn_ref[0, 0]
    aidx = aidx_ref[0]
    w1x = w1x_ref[...]
    w1f = w1f_ref[...]
    w1a = w1a_ref[...]
    b1 = b1_ref[...]
    w2 = w2_ref[...]
    b2 = b2_ref[...]

    iota = lax.broadcasted_iota(jnp.int32, (NP, N), 1).astype(F32)
    oh_a = (iota == aidx).astype(F32)

    ax = jnp.sum(jnp.where(iota == aidx, xa[0:1, :], 0.0), axis=1, keepdims=True)
    ay = jnp.sum(jnp.where(iota == aidx, xa[1:2, :], 0.0), axis=1, keepdims=True)
    az = jnp.sum(jnp.where(iota == aidx, xa[2:3, :], 0.0), axis=1, keepdims=True)
    nxyz_ref[0, 0] = jnp.concatenate([ax, ay, az], axis=1)

    xyz1 = jnp.transpose(xa)
    xyz2 = jnp.transpose(xn)
    dot = functools.partial(jnp.dot, preferred_element_type=F32)
    q1 = dot(f1, w1a) - dot(xyz1, w1x)
    a_mat = dot(oh_a, q1) + b1
    p_mat = dot(f2, w1f) + dot(xyz2, w1x)

    dx = ax - xn[0:1, :]
    dy = ay - xn[1:2, :]
    dz = az - xn[2:3, :]
    s = (dx * dx + dy * dy) + dz * dz

    acc = None
    for _ in range(K):
        mval = jnp.min(s, axis=1, keepdims=True)
        nidx = jnp.min(jnp.where(s == mval, iota, F32(1e9)), axis=1, keepdims=True)
        oh = (iota == nidx).astype(F32)
        g = dot(oh, p_mat)
        h = jnp.maximum(g + a_mat, 0.0)
        o = jnp.maximum(dot(h, w2) + b2, 0.0)
        acc = o if acc is None else jnp.maximum(acc, o)
        s = jnp.where(iota == nidx, F32(3e38), s)

    nfeat_ref[0, 0] = acc


def kernel(xyzs, features, W1, b1, W2, b2):
    aidx = _run_fps(xyzs)
    aidx3 = aidx.reshape(B * T, NP, 1)

    xt = jnp.transpose(xyzs, (0, 1, 3, 2))
    w1x = W1[0:3]
    w1f = W1[3:3 + C]
    w1a = W1[3 + C:3 + 2 * C]
    b1r = b1.reshape(1, C)
    b2r = b2.reshape(1, 2 * C)

    t_last = T - 1
    nb = lambda b, t: (b, jnp.minimum(t + 1, t_last), 0, 0)

    out_shapes = (
        jax.ShapeDtypeStruct((B, T, NP, 3), F32),
        jax.ShapeDtypeStruct((B, T, NP, 2 * C), F32),
    )
    grid = (B, T)
    new_xyzs, new_feats = pl.pallas_call(
        _main_body,
        grid=grid,
        out_shape=out_shapes,
        in_specs=[
            pl.BlockSpec((1, 1, 3, N), lambda b, t: (b, t, 0, 0)),
            pl.BlockSpec((1, 1, 3, N), nb),
            pl.BlockSpec((1, 1, N, C), lambda b, t: (b, t, 0, 0)),
            pl.BlockSpec((1, 1, N, C), nb),
            pl.BlockSpec((1, NP, 1), lambda b, t: (b * T + t, 0, 0)),
            pl.BlockSpec((3, C), lambda b, t: (0, 0)),
            pl.BlockSpec((C, C), lambda b, t: (0, 0)),
            pl.BlockSpec((C, C), lambda b, t: (0, 0)),
            pl.BlockSpec((1, C), lambda b, t: (0, 0)),
            pl.BlockSpec((C, 2 * C), lambda b, t: (0, 0)),
            pl.BlockSpec((1, 2 * C), lambda b, t: (0, 0)),
        ],
        out_specs=(
            pl.BlockSpec((1, 1, NP, 3), lambda b, t: (b, t, 0, 0)),
            pl.BlockSpec((1, 1, NP, 2 * C), lambda b, t: (b, t, 0, 0)),
        ),
    )(xt, xt, features, features, aidx3, w1x, w1f, w1a, b1r, W2, b2r)
    return new_xyzs, new_feats
